# Initial kernel scaffold; baseline (speedup 1.0000x reference)
import jax
import jax.numpy as jnp
from jax.experimental import pallas as pl


def kernel(x, edge_index, W1, b1, W2, b2, W3, b3):
    raise NotImplementedError("write your pallas kernel here")


# speedup vs baseline: 25.6466x; 25.6466x over previous
import functools

import jax
import jax.numpy as jnp
from jax import lax
from jax.experimental import pallas as pl
from jax.experimental.pallas import tpu as pltpu
from jax.experimental.pallas import tpu_sc as plsc

N = 10000
E = 320000
D_IN = 128

NC = 2
NS = 16
NW = NC * NS

G = 128
K = (E + NW * G - 1) // (NW * G)
E_PAD = NW * K * G
N_PAD = 10240
RPT = N_PAD // NS

BR = 1024


def _sc_mesh():
    return plsc.VectorSubcoreMesh(core_axis_name="c", subcore_axis_name="s")


def _make_sc_hist():
    @functools.partial(
        pl.kernel,
        out_type=jax.ShapeDtypeStruct((NC, N_PAD, 16), jnp.float32),
        mesh=_sc_mesh(),
        compiler_params=pltpu.CompilerParams(use_tc_tiling_on_sc=False),
        scratch_types=[
            pltpu.VMEM((K, G), jnp.int32),
            pltpu.VMEM((G, 16), jnp.float32),
            pltpu.VMEM((G, 16), jnp.float32),
            pltpu.VMEM_SHARED((N_PAD, 16), jnp.float32),
        ],
    )
    def hist(dst_hbm, ones_hbm, zeros_hbm, out_hbm, dst_v, ones_v, zeros_v, acc_sh):
        c = lax.axis_index("c")
        s = lax.axis_index("s")
        w = s * NC + c
        pltpu.sync_copy(zeros_hbm, zeros_v)
        pltpu.sync_copy(dst_hbm.at[w], dst_v)

        @pl.loop(0, RPT // G)
        def _(t):
            pltpu.sync_copy(zeros_v, acc_sh.at[pl.ds(s * RPT + t * G, G)])

        plsc.subcore_barrier()

        @pl.loop(0, K)
        def _(j):
            pltpu.sync_copy(ones_hbm, ones_v)
            pltpu.sync_copy(ones_v, acc_sh.at[dst_v.at[j]], add=True)

        plsc.subcore_barrier()
        pltpu.sync_copy(acc_sh.at[pl.ds(s * RPT, RPT)],
                        out_hbm.at[c, pl.ds(s * RPT, RPT)])

    return hist


def _make_sc_agg(F):
    @functools.partial(
        pl.kernel,
        out_type=jax.ShapeDtypeStruct((NC, N_PAD, F), jnp.float32),
        mesh=_sc_mesh(),
        compiler_params=pltpu.CompilerParams(use_tc_tiling_on_sc=False),
        scratch_types=[
            pltpu.VMEM((K, G), jnp.int32),
            pltpu.VMEM((K, G), jnp.int32),
            pltpu.VMEM((G, F), jnp.float32),
            pltpu.VMEM((G, F), jnp.float32),
            pltpu.VMEM_SHARED((N_PAD, F), jnp.float32),
        ],
    )
    def agg(g_hbm, src_hbm, dst_hbm, zeros_hbm, out_hbm,
            src_v, dst_v, rows_v, zeros_v, acc_sh):
        c = lax.axis_index("c")
        s = lax.axis_index("s")
        w = s * NC + c
        pltpu.sync_copy(zeros_hbm, zeros_v)
        pltpu.sync_copy(src_hbm.at[w], src_v)
        pltpu.sync_copy(dst_hbm.at[w], dst_v)

        @pl.loop(0, RPT // G)
        def _(t):
            pltpu.sync_copy(zeros_v, acc_sh.at[pl.ds(s * RPT + t * G, G)])

        plsc.subcore_barrier()

        @pl.loop(0, K)
        def _(j):
            pltpu.sync_copy(g_hbm.at[src_v.at[j]], rows_v)
            pltpu.sync_copy(rows_v, acc_sh.at[dst_v.at[j]], add=True)

        plsc.subcore_barrier()
        pltpu.sync_copy(acc_sh.at[pl.ds(s * RPT, RPT)],
                        out_hbm.at[c, pl.ds(s * RPT, RPT)])

    return agg


_sc_hist = _make_sc_hist()
_sc_agg32 = _make_sc_agg(32)
_sc_agg16 = _make_sc_agg(16)


_HI = jax.lax.Precision.HIGHEST


def _mm1_body(x_ref, w_ref, o_ref):
    o_ref[...] = jnp.dot(x_ref[...], w_ref[...],
                         preferred_element_type=jnp.float32, precision=_HI)


def _tc_mm1(x_pad, W1):
    return pl.pallas_call(
        _mm1_body,
        out_shape=jax.ShapeDtypeStruct((N_PAD, 32), jnp.float32),
        grid=(N_PAD // BR,),
        in_specs=[pl.BlockSpec((BR, D_IN), lambda i: (i, 0)),
                  pl.BlockSpec((D_IN, 32), lambda i: (0, 0))],
        out_specs=pl.BlockSpec((BR, 32), lambda i: (i, 0)),
    )(x_pad, W1)


def _prep_body(cnt_ref, m1_ref, d_ref, g1_ref):
    cnt = cnt_ref[0, :, 0:1] + cnt_ref[1, :, 0:1]
    d = lax.rsqrt(cnt + 1.0)
    d_ref[...] = d
    g1_ref[...] = d * m1_ref[...]


def _tc_prep(cnt2, m1):
    return pl.pallas_call(
        _prep_body,
        out_shape=[jax.ShapeDtypeStruct((N_PAD, 1), jnp.float32),
                   jax.ShapeDtypeStruct((N_PAD, 32), jnp.float32)],
        grid=(N_PAD // BR,),
        in_specs=[pl.BlockSpec((NC, BR, 16), lambda i: (0, i, 0)),
                  pl.BlockSpec((BR, 32), lambda i: (i, 0))],
        out_specs=[pl.BlockSpec((BR, 1), lambda i: (i, 0)),
                   pl.BlockSpec((BR, 32), lambda i: (i, 0))],
    )(cnt2, m1)


def _l2_body(a_ref, m1_ref, d_ref, w2_ref, b1_ref, o_ref):
    d = d_ref[...]
    acc = a_ref[0] + a_ref[1]
    h1 = jnp.maximum(d * acc + (d * d) * m1_ref[...] + b1_ref[...], 0.0)
    o_ref[...] = jnp.dot(d * h1, w2_ref[...],
                         preferred_element_type=jnp.float32, precision=_HI)


def _tc_layer2(acc1, m1, d, W2, b1r):
    return pl.pallas_call(
        _l2_body,
        out_shape=jax.ShapeDtypeStruct((N_PAD, 16), jnp.float32),
        grid=(N_PAD // BR,),
        in_specs=[pl.BlockSpec((NC, BR, 32), lambda i: (0, i, 0)),
                  pl.BlockSpec((BR, 32), lambda i: (i, 0)),
                  pl.BlockSpec((BR, 1), lambda i: (i, 0)),
                  pl.BlockSpec((32, 16), lambda i: (0, 0)),
                  pl.BlockSpec((1, 32), lambda i: (0, 0))],
        out_specs=pl.BlockSpec((BR, 16), lambda i: (i, 0)),
    )(acc1, m1, d, W2, b1r)


def _l3_body(a_ref, g2_ref, d_ref, b2_ref, o_ref):
    d = d_ref[...]
    acc = a_ref[0] + a_ref[1]
    h2 = jnp.maximum(d * acc + d * g2_ref[...] + b2_ref[...], 0.0)
    o_ref[...] = d * h2


def _tc_layer3(acc2, g2, d, b2r):
    return pl.pallas_call(
        _l3_body,
        out_shape=jax.ShapeDtypeStruct((N_PAD, 16), jnp.float32),
        grid=(N_PAD // BR,),
        in_specs=[pl.BlockSpec((NC, BR, 16), lambda i: (0, i, 0)),
                  pl.BlockSpec((BR, 16), lambda i: (i, 0)),
                  pl.BlockSpec((BR, 1), lambda i: (i, 0)),
                  pl.BlockSpec((1, 16), lambda i: (0, 0))],
        out_specs=pl.BlockSpec((BR, 16), lambda i: (i, 0)),
    )(acc2, g2, d, b2r)


def _final_body(a_ref, q_ref, d_ref, w3_ref, b3_ref, o_ref):
    d = d_ref[...]
    m = d * (a_ref[0] + a_ref[1] + q_ref[...])
    o_ref[...] = jnp.dot(m, w3_ref[...],
                         preferred_element_type=jnp.float32,
                         precision=_HI) + b3_ref[...]


def _tc_final(acc3, q, d, W3, b3r):
    return pl.pallas_call(
        _final_body,
        out_shape=jax.ShapeDtypeStruct((N_PAD, 1), jnp.float32),
        grid=(N_PAD // BR,),
        in_specs=[pl.BlockSpec((NC, BR, 16), lambda i: (0, i, 0)),
                  pl.BlockSpec((BR, 16), lambda i: (i, 0)),
                  pl.BlockSpec((BR, 1), lambda i: (i, 0)),
                  pl.BlockSpec((16, 1), lambda i: (0, 0)),
                  pl.BlockSpec((1, 1), lambda i: (0, 0))],
        out_specs=pl.BlockSpec((BR, 1), lambda i: (i, 0)),
    )(acc3, q, d, W3, b3r)


@jax.jit
def kernel(x, edge_index, W1, b1, W2, b2, W3, b3):
    x_pad = jnp.pad(x, ((0, N_PAD - N), (0, 0)))
    pad_e = E_PAD - E
    ar = jnp.arange(pad_e, dtype=jnp.int32)
    pad_src = ar % 16
    pad_dst = N + (ar % (N_PAD - N))
    src3 = jnp.concatenate([edge_index[0], pad_src]).reshape(NW, K, G)
    dst3 = jnp.concatenate([edge_index[1], pad_dst]).reshape(NW, K, G)

    ones16 = jnp.ones((G, 16), jnp.float32)
    zeros16 = jnp.zeros((G, 16), jnp.float32)
    zeros32 = jnp.zeros((G, 32), jnp.float32)
    b1r = b1.reshape(1, 32)
    b2r = b2.reshape(1, 16)
    b3r = b3.reshape(1, 1)

    cnt2 = _sc_hist(dst3, ones16, zeros16)
    m1 = _tc_mm1(x_pad, W1)

    d, g1 = _tc_prep(cnt2, m1)

    acc1 = _sc_agg32(g1, src3, dst3, zeros32)
    g2 = _tc_layer2(acc1, m1, d, W2, b1r)

    acc2 = _sc_agg16(g2, src3, dst3, zeros16)
    q = _tc_layer3(acc2, g2, d, b2r)

    acc3 = _sc_agg16(q, src3, dst3, zeros16)
    out = _tc_final(acc3, q, d, W3, b3r)

    return out[:N]


# speedup vs baseline: 33.9559x; 1.3240x over previous
import functools

import jax
import jax.numpy as jnp
from jax import lax
from jax.experimental import pallas as pl
from jax.experimental.pallas import tpu as pltpu
from jax.experimental.pallas import tpu_sc as plsc

N = 10000
E = 320000
D_IN = 128

NC = 2
NS = 16
NW = NC * NS

G = 128
K = (E + NW * G - 1) // (NW * G)
E_PAD = NW * K * G
N_PAD = 10240
RPT = N_PAD // NS

BR = 1024


def _sc_mesh():
    return plsc.VectorSubcoreMesh(core_axis_name="c", subcore_axis_name="s")


def _make_sc_hist():
    @functools.partial(
        pl.kernel,
        out_type=jax.ShapeDtypeStruct((NC, N_PAD, 16), jnp.float32),
        mesh=_sc_mesh(),
        compiler_params=pltpu.CompilerParams(use_tc_tiling_on_sc=False),
        scratch_types=[
            pltpu.VMEM((K, G), jnp.int32),
            pltpu.VMEM((G, 16), jnp.float32),
            pltpu.VMEM((G, 16), jnp.float32),
            pltpu.VMEM_SHARED((N_PAD, 16), jnp.float32),
        ],
    )
    def hist(dst_hbm, ones_hbm, zeros_hbm, out_hbm, dst_v, ones_v, zeros_v, acc_sh):
        c = lax.axis_index("c")
        s = lax.axis_index("s")
        w = s * NC + c
        pltpu.sync_copy(zeros_hbm, zeros_v)
        pltpu.sync_copy(ones_hbm, ones_v)
        pltpu.sync_copy(dst_hbm.at[w], dst_v)

        @pl.loop(0, RPT // G)
        def _(t):
            pltpu.sync_copy(zeros_v, acc_sh.at[pl.ds(s * RPT + t * G, G)])

        plsc.subcore_barrier()

        @pl.loop(0, K)
        def _(j):
            pltpu.sync_copy(ones_hbm, ones_v)
            pltpu.sync_copy(ones_v, acc_sh.at[dst_v.at[j]], add=True)

        plsc.subcore_barrier()
        pltpu.sync_copy(acc_sh.at[pl.ds(s * RPT, RPT)],
                        out_hbm.at[c, pl.ds(s * RPT, RPT)])

    return hist


def _make_sc_agg(F):
    @functools.partial(
        pl.kernel,
        out_type=jax.ShapeDtypeStruct((NC, N_PAD, F), jnp.float32),
        mesh=_sc_mesh(),
        compiler_params=pltpu.CompilerParams(use_tc_tiling_on_sc=False),
        scratch_types=[
            pltpu.VMEM((K, G), jnp.int32),
            pltpu.VMEM((K, G), jnp.int32),
            pltpu.VMEM((G, F), jnp.float32),
            pltpu.VMEM((G, F), jnp.float32),
            pltpu.VMEM_SHARED((N_PAD, F), jnp.float32),
            pltpu.VMEM_SHARED((N_PAD, F), jnp.float32),
        ],
    )
    def agg(g_hbm, src_hbm, dst_hbm, zeros_hbm, out_hbm,
            src_v, dst_v, rows_v, zeros_v, g_sh, acc_sh):
        c = lax.axis_index("c")
        s = lax.axis_index("s")
        w = s * NC + c
        pltpu.sync_copy(zeros_hbm, zeros_v)
        pltpu.sync_copy(g_hbm.at[pl.ds(s * RPT, RPT)],
                        g_sh.at[pl.ds(s * RPT, RPT)])
        pltpu.sync_copy(src_hbm.at[w], src_v)
        pltpu.sync_copy(dst_hbm.at[w], dst_v)

        @pl.loop(0, RPT // G)
        def _(t):
            pltpu.sync_copy(zeros_v, acc_sh.at[pl.ds(s * RPT + t * G, G)])

        plsc.subcore_barrier()

        @pl.loop(0, K)
        def _(j):
            pltpu.sync_copy(g_sh.at[src_v.at[j]], rows_v)
            pltpu.sync_copy(rows_v, acc_sh.at[dst_v.at[j]], add=True)

        plsc.subcore_barrier()
        pltpu.sync_copy(acc_sh.at[pl.ds(s * RPT, RPT)],
                        out_hbm.at[c, pl.ds(s * RPT, RPT)])

    return agg


_sc_hist = _make_sc_hist()
_sc_agg32 = _make_sc_agg(32)
_sc_agg16 = _make_sc_agg(16)


_HI = jax.lax.Precision.HIGHEST


def _mm1_body(x_ref, w_ref, o_ref):
    o_ref[...] = jnp.dot(x_ref[...], w_ref[...],
                         preferred_element_type=jnp.float32, precision=_HI)


def _tc_mm1(x_pad, W1):
    return pl.pallas_call(
        _mm1_body,
        out_shape=jax.ShapeDtypeStruct((N_PAD, 32), jnp.float32),
        grid=(N_PAD // BR,),
        in_specs=[pl.BlockSpec((BR, D_IN), lambda i: (i, 0)),
                  pl.BlockSpec((D_IN, 32), lambda i: (0, 0))],
        out_specs=pl.BlockSpec((BR, 32), lambda i: (i, 0)),
    )(x_pad, W1)


def _prep_body(cnt_ref, m1_ref, d_ref, g1_ref):
    cnt = cnt_ref[0, :, 0:1] + cnt_ref[1, :, 0:1]
    d = lax.rsqrt(cnt + 1.0)
    d_ref[...] = d
    g1_ref[...] = d * m1_ref[...]


def _tc_prep(cnt2, m1):
    return pl.pallas_call(
        _prep_body,
        out_shape=[jax.ShapeDtypeStruct((N_PAD, 1), jnp.float32),
                   jax.ShapeDtypeStruct((N_PAD, 32), jnp.float32)],
        grid=(N_PAD // BR,),
        in_specs=[pl.BlockSpec((NC, BR, 16), lambda i: (0, i, 0)),
                  pl.BlockSpec((BR, 32), lambda i: (i, 0))],
        out_specs=[pl.BlockSpec((BR, 1), lambda i: (i, 0)),
                   pl.BlockSpec((BR, 32), lambda i: (i, 0))],
    )(cnt2, m1)


def _l2_body(a_ref, m1_ref, d_ref, w2_ref, b1_ref, o_ref):
    d = d_ref[...]
    acc = a_ref[0] + a_ref[1]
    h1 = jnp.maximum(d * acc + (d * d) * m1_ref[...] + b1_ref[...], 0.0)
    o_ref[...] = jnp.dot(d * h1, w2_ref[...],
                         preferred_element_type=jnp.float32, precision=_HI)


def _tc_layer2(acc1, m1, d, W2, b1r):
    return pl.pallas_call(
        _l2_body,
        out_shape=jax.ShapeDtypeStruct((N_PAD, 16), jnp.float32),
        grid=(N_PAD // BR,),
        in_specs=[pl.BlockSpec((NC, BR, 32), lambda i: (0, i, 0)),
                  pl.BlockSpec((BR, 32), lambda i: (i, 0)),
                  pl.BlockSpec((BR, 1), lambda i: (i, 0)),
                  pl.BlockSpec((32, 16), lambda i: (0, 0)),
                  pl.BlockSpec((1, 32), lambda i: (0, 0))],
        out_specs=pl.BlockSpec((BR, 16), lambda i: (i, 0)),
    )(acc1, m1, d, W2, b1r)


def _l3_body(a_ref, g2_ref, d_ref, b2_ref, o_ref):
    d = d_ref[...]
    acc = a_ref[0] + a_ref[1]
    h2 = jnp.maximum(d * acc + d * g2_ref[...] + b2_ref[...], 0.0)
    o_ref[...] = d * h2


def _tc_layer3(acc2, g2, d, b2r):
    return pl.pallas_call(
        _l3_body,
        out_shape=jax.ShapeDtypeStruct((N_PAD, 16), jnp.float32),
        grid=(N_PAD // BR,),
        in_specs=[pl.BlockSpec((NC, BR, 16), lambda i: (0, i, 0)),
                  pl.BlockSpec((BR, 16), lambda i: (i, 0)),
                  pl.BlockSpec((BR, 1), lambda i: (i, 0)),
                  pl.BlockSpec((1, 16), lambda i: (0, 0))],
        out_specs=pl.BlockSpec((BR, 16), lambda i: (i, 0)),
    )(acc2, g2, d, b2r)


def _final_body(a_ref, q_ref, d_ref, w3_ref, b3_ref, o_ref):
    d = d_ref[...]
    m = d * (a_ref[0] + a_ref[1] + q_ref[...])
    o_ref[...] = jnp.dot(m, w3_ref[...],
                         preferred_element_type=jnp.float32,
                         precision=_HI) + b3_ref[...]


def _tc_final(acc3, q, d, W3, b3r):
    return pl.pallas_call(
        _final_body,
        out_shape=jax.ShapeDtypeStruct((N_PAD, 1), jnp.float32),
        grid=(N_PAD // BR,),
        in_specs=[pl.BlockSpec((NC, BR, 16), lambda i: (0, i, 0)),
                  pl.BlockSpec((BR, 16), lambda i: (i, 0)),
                  pl.BlockSpec((BR, 1), lambda i: (i, 0)),
                  pl.BlockSpec((16, 1), lambda i: (0, 0)),
                  pl.BlockSpec((1, 1), lambda i: (0, 0))],
        out_specs=pl.BlockSpec((BR, 1), lambda i: (i, 0)),
    )(acc3, q, d, W3, b3r)


@jax.jit
def kernel(x, edge_index, W1, b1, W2, b2, W3, b3):
    x_pad = jnp.pad(x, ((0, N_PAD - N), (0, 0)))
    pad_e = E_PAD - E
    ar = jnp.arange(pad_e, dtype=jnp.int32)
    pad_src = ar % 16
    pad_dst = N + (ar % (N_PAD - N))
    src3 = jnp.concatenate([edge_index[0], pad_src]).reshape(NW, K, G)
    dst3 = jnp.concatenate([edge_index[1], pad_dst]).reshape(NW, K, G)

    ones16 = jnp.ones((G, 16), jnp.float32)
    zeros16 = jnp.zeros((G, 16), jnp.float32)
    zeros32 = jnp.zeros((G, 32), jnp.float32)
    b1r = b1.reshape(1, 32)
    b2r = b2.reshape(1, 16)
    b3r = b3.reshape(1, 1)

    cnt2 = _sc_hist(dst3, ones16, zeros16)
    m1 = _tc_mm1(x_pad, W1)

    d, g1 = _tc_prep(cnt2, m1)

    acc1 = _sc_agg32(g1, src3, dst3, zeros32)
    g2 = _tc_layer2(acc1, m1, d, W2, b1r)

    acc2 = _sc_agg16(g2, src3, dst3, zeros16)
    q = _tc_layer3(acc2, g2, d, b2r)

    acc3 = _sc_agg16(q, src3, dst3, zeros16)
    out = _tc_final(acc3, q, d, W3, b3r)

    return out[:N]


# speedup vs baseline: 42.5522x; 1.2532x over previous
import functools

import jax
import jax.numpy as jnp
from jax import lax
from jax.experimental import pallas as pl
from jax.experimental.pallas import tpu as pltpu
from jax.experimental.pallas import tpu_sc as plsc

N = 10000
E = 320000
D_IN = 128

NC = 2
NS = 16
NW = NC * NS

G = 128
K = (E + NW * G - 1) // (NW * G)
E_PAD = NW * K * G
N_PAD = 10240
RPT = N_PAD // NS

BR = 1024


def _sc_mesh():
    return plsc.VectorSubcoreMesh(core_axis_name="c", subcore_axis_name="s")


def _make_sc_hist():
    @functools.partial(
        pl.kernel,
        out_type=jax.ShapeDtypeStruct((NC, N_PAD, 16), jnp.float32),
        mesh=_sc_mesh(),
        compiler_params=pltpu.CompilerParams(use_tc_tiling_on_sc=False),
        scratch_types=[
            pltpu.VMEM((K, G), jnp.int32),
            pltpu.VMEM((G, 16), jnp.float32),
            pltpu.VMEM((G, 16), jnp.float32),
            pltpu.VMEM_SHARED((G, 16), jnp.float32),
            pltpu.VMEM_SHARED((N_PAD, 16), jnp.float32),
        ],
    )
    def hist(dst_hbm, ones_hbm, zeros_hbm, out_hbm, dst_v, ones_v, zeros_v,
             ones_sh, acc_sh):
        c = lax.axis_index("c")
        s = lax.axis_index("s")
        w = s * NC + c
        pltpu.sync_copy(zeros_hbm, zeros_v)
        pltpu.sync_copy(ones_hbm, ones_v)

        @pl.when(s == 0)
        def _():
            pltpu.sync_copy(ones_v, ones_sh)

        pltpu.sync_copy(dst_hbm.at[w], dst_v)

        @pl.loop(0, RPT // G)
        def _(t):
            pltpu.sync_copy(zeros_v, acc_sh.at[pl.ds(s * RPT + t * G, G)])

        plsc.subcore_barrier()

        @pl.loop(0, K)
        def _(j):
            pltpu.sync_copy(ones_sh, ones_v)
            pltpu.sync_copy(ones_v, acc_sh.at[dst_v.at[j]], add=True)

        plsc.subcore_barrier()
        pltpu.sync_copy(acc_sh.at[pl.ds(s * RPT, RPT)],
                        out_hbm.at[c, pl.ds(s * RPT, RPT)])

    return hist


def _make_sc_agg(F):
    @functools.partial(
        pl.kernel,
        out_type=jax.ShapeDtypeStruct((NC, N_PAD, F), jnp.float32),
        mesh=_sc_mesh(),
        compiler_params=pltpu.CompilerParams(use_tc_tiling_on_sc=False),
        scratch_types=[
            pltpu.VMEM((K, G), jnp.int32),
            pltpu.VMEM((K, G), jnp.int32),
            pltpu.VMEM((G, F), jnp.float32),
            pltpu.VMEM((G, F), jnp.float32),
            pltpu.VMEM_SHARED((N_PAD, F), jnp.float32),
            pltpu.VMEM_SHARED((N_PAD, F), jnp.float32),
        ],
    )
    def agg(g_hbm, src_hbm, dst_hbm, zeros_hbm, out_hbm,
            src_v, dst_v, rows_v, zeros_v, g_sh, acc_sh):
        c = lax.axis_index("c")
        s = lax.axis_index("s")
        w = s * NC + c
        pltpu.sync_copy(zeros_hbm, zeros_v)
        pltpu.sync_copy(g_hbm.at[pl.ds(s * RPT, RPT)],
                        g_sh.at[pl.ds(s * RPT, RPT)])
        pltpu.sync_copy(src_hbm.at[w], src_v)
        pltpu.sync_copy(dst_hbm.at[w], dst_v)

        @pl.loop(0, RPT // G)
        def _(t):
            pltpu.sync_copy(zeros_v, acc_sh.at[pl.ds(s * RPT + t * G, G)])

        plsc.subcore_barrier()

        @pl.loop(0, K)
        def _(j):
            pltpu.sync_copy(g_sh.at[src_v.at[j]], rows_v)
            pltpu.sync_copy(rows_v, acc_sh.at[dst_v.at[j]], add=True)

        plsc.subcore_barrier()
        pltpu.sync_copy(acc_sh.at[pl.ds(s * RPT, RPT)],
                        out_hbm.at[c, pl.ds(s * RPT, RPT)])

    return agg


_sc_hist = _make_sc_hist()
_sc_agg32 = _make_sc_agg(32)
_sc_agg16 = _make_sc_agg(16)


_HI = jax.lax.Precision.HIGHEST


def _mm1_body(x_ref, w_ref, o_ref):
    o_ref[...] = jnp.dot(x_ref[...], w_ref[...],
                         preferred_element_type=jnp.float32, precision=_HI)


def _tc_mm1(x_pad, W1):
    return pl.pallas_call(
        _mm1_body,
        out_shape=jax.ShapeDtypeStruct((N_PAD, 32), jnp.float32),
        grid=(N_PAD // BR,),
        in_specs=[pl.BlockSpec((BR, D_IN), lambda i: (i, 0)),
                  pl.BlockSpec((D_IN, 32), lambda i: (0, 0))],
        out_specs=pl.BlockSpec((BR, 32), lambda i: (i, 0)),
    )(x_pad, W1)


def _prep_body(cnt_ref, m1_ref, d_ref, g1_ref):
    cnt = cnt_ref[0, :, 0:1] + cnt_ref[1, :, 0:1]
    d = lax.rsqrt(cnt + 1.0)
    d_ref[...] = d
    g1_ref[...] = d * m1_ref[...]


def _tc_prep(cnt2, m1):
    return pl.pallas_call(
        _prep_body,
        out_shape=[jax.ShapeDtypeStruct((N_PAD, 1), jnp.float32),
                   jax.ShapeDtypeStruct((N_PAD, 32), jnp.float32)],
        grid=(N_PAD // BR,),
        in_specs=[pl.BlockSpec((NC, BR, 16), lambda i: (0, i, 0)),
                  pl.BlockSpec((BR, 32), lambda i: (i, 0))],
        out_specs=[pl.BlockSpec((BR, 1), lambda i: (i, 0)),
                   pl.BlockSpec((BR, 32), lambda i: (i, 0))],
    )(cnt2, m1)


def _l2_body(a_ref, m1_ref, d_ref, w2_ref, b1_ref, o_ref):
    d = d_ref[...]
    acc = a_ref[0] + a_ref[1]
    h1 = jnp.maximum(d * acc + (d * d) * m1_ref[...] + b1_ref[...], 0.0)
    o_ref[...] = jnp.dot(d * h1, w2_ref[...],
                         preferred_element_type=jnp.float32, precision=_HI)


def _tc_layer2(acc1, m1, d, W2, b1r):
    return pl.pallas_call(
        _l2_body,
        out_shape=jax.ShapeDtypeStruct((N_PAD, 16), jnp.float32),
        grid=(N_PAD // BR,),
        in_specs=[pl.BlockSpec((NC, BR, 32), lambda i: (0, i, 0)),
                  pl.BlockSpec((BR, 32), lambda i: (i, 0)),
                  pl.BlockSpec((BR, 1), lambda i: (i, 0)),
                  pl.BlockSpec((32, 16), lambda i: (0, 0)),
                  pl.BlockSpec((1, 32), lambda i: (0, 0))],
        out_specs=pl.BlockSpec((BR, 16), lambda i: (i, 0)),
    )(acc1, m1, d, W2, b1r)


def _l3_body(a_ref, g2_ref, d_ref, b2_ref, o_ref):
    d = d_ref[...]
    acc = a_ref[0] + a_ref[1]
    h2 = jnp.maximum(d * acc + d * g2_ref[...] + b2_ref[...], 0.0)
    o_ref[...] = d * h2


def _tc_layer3(acc2, g2, d, b2r):
    return pl.pallas_call(
        _l3_body,
        out_shape=jax.ShapeDtypeStruct((N_PAD, 16), jnp.float32),
        grid=(N_PAD // BR,),
        in_specs=[pl.BlockSpec((NC, BR, 16), lambda i: (0, i, 0)),
                  pl.BlockSpec((BR, 16), lambda i: (i, 0)),
                  pl.BlockSpec((BR, 1), lambda i: (i, 0)),
                  pl.BlockSpec((1, 16), lambda i: (0, 0))],
        out_specs=pl.BlockSpec((BR, 16), lambda i: (i, 0)),
    )(acc2, g2, d, b2r)


def _final_body(a_ref, q_ref, d_ref, w3_ref, b3_ref, o_ref):
    d = d_ref[...]
    m = d * (a_ref[0] + a_ref[1] + q_ref[...])
    o_ref[...] = jnp.dot(m, w3_ref[...],
                         preferred_element_type=jnp.float32,
                         precision=_HI) + b3_ref[...]


def _tc_final(acc3, q, d, W3, b3r):
    return pl.pallas_call(
        _final_body,
        out_shape=jax.ShapeDtypeStruct((N_PAD, 1), jnp.float32),
        grid=(N_PAD // BR,),
        in_specs=[pl.BlockSpec((NC, BR, 16), lambda i: (0, i, 0)),
                  pl.BlockSpec((BR, 16), lambda i: (i, 0)),
                  pl.BlockSpec((BR, 1), lambda i: (i, 0)),
                  pl.BlockSpec((16, 1), lambda i: (0, 0)),
                  pl.BlockSpec((1, 1), lambda i: (0, 0))],
        out_specs=pl.BlockSpec((BR, 1), lambda i: (i, 0)),
    )(acc3, q, d, W3, b3r)


@jax.jit
def kernel(x, edge_index, W1, b1, W2, b2, W3, b3):
    x_pad = jnp.pad(x, ((0, N_PAD - N), (0, 0)))
    pad_e = E_PAD - E
    ar = jnp.arange(pad_e, dtype=jnp.int32)
    pad_src = ar % 16
    pad_dst = N + (ar % (N_PAD - N))
    src3 = jnp.concatenate([edge_index[0], pad_src]).reshape(NW, K, G)
    dst3 = jnp.concatenate([edge_index[1], pad_dst]).reshape(NW, K, G)

    ones16 = jnp.ones((G, 16), jnp.float32)
    zeros16 = jnp.zeros((G, 16), jnp.float32)
    zeros32 = jnp.zeros((G, 32), jnp.float32)
    b1r = b1.reshape(1, 32)
    b2r = b2.reshape(1, 16)
    b3r = b3.reshape(1, 1)

    cnt2 = _sc_hist(dst3, ones16, zeros16)
    m1 = _tc_mm1(x_pad, W1)

    d, g1 = _tc_prep(cnt2, m1)

    acc1 = _sc_agg32(g1, src3, dst3, zeros32)
    g2 = _tc_layer2(acc1, m1, d, W2, b1r)

    acc2 = _sc_agg16(g2, src3, dst3, zeros16)
    q = _tc_layer3(acc2, g2, d, b2r)

    acc3 = _sc_agg16(q, src3, dst3, zeros16)
    out = _tc_final(acc3, q, d, W3, b3r)

    return out[:N]
